# baseline (speedup 1.0000x reference)
import functools

import jax
import jax.numpy as jnp
from jax import lax
from jax.experimental import pallas as pl
from jax.experimental.pallas import tpu as pltpu
from jax.experimental.pallas import tpu_sc as plsc

_B = 16384
_D = 64
_EPS = 1e-5
_NW = 32
_BPW = _B // _NW
_CHUNK = 128
_NCHUNK = _BPW // _CHUNK
_NC = _D // 16


def _lanesum(x):
    lanes = lax.iota(jnp.int32, 16)
    for sh in (8, 4, 2, 1):
        x = x + x.at[lanes ^ sh].get(mode="promise_in_bounds")
    return x


def _rsqrt(v):
    i = lax.bitcast_convert_type(v, jnp.int32)
    i = jnp.int32(0x5F3759DF) - lax.shift_right_arithmetic(i, 1)
    y = lax.bitcast_convert_type(i, jnp.float32)
    for _ in range(3):
        y = y * (jnp.float32(1.5) - jnp.float32(0.5) * v * y * y)
    return y


def _body(x_hbm, table_hbm, gamma_hbm, beta_hbm, out_hbm,
          idx_v, rows_v, gam_v, bet_v, sem):
    wid = lax.axis_index("s") * 2 + lax.axis_index("c")
    base = wid * _BPW

    for j in range(_NCHUNK):
        pltpu.sync_copy(x_hbm.at[pl.ds(base + j * _CHUNK, _CHUNK)],
                        idx_v.at[j])
    pltpu.sync_copy(gamma_hbm, gam_v)
    pltpu.sync_copy(beta_hbm, bet_v)

    copies = [
        pltpu.async_copy(table_hbm.at[idx_v.at[j]],
                         rows_v.at[pl.ds(j * _CHUNK, _CHUNK)], sem)
        for j in range(_NCHUNK)
    ]
    for c in copies:
        c.wait()

    g = [gam_v[pl.ds(c * 16, 16)] for c in range(_NC)]
    b = [bet_v[pl.ds(c * 16, 16)] for c in range(_NC)]
    inv_d = jnp.float32(1.0 / _D)

    def row(r, carry):
        v = [rows_v[r, pl.ds(c * 16, 16)] for c in range(_NC)]
        s = (v[0] + v[1]) + (v[2] + v[3])
        q = (v[0] * v[0] + v[1] * v[1]) + (v[2] * v[2] + v[3] * v[3])
        mean = _lanesum(s) * inv_d
        ex2 = _lanesum(q) * inv_d
        var = ex2 - mean * mean + jnp.float32(_EPS)
        rs = _rsqrt(var)
        for c in range(_NC):
            rows_v[r, pl.ds(c * 16, 16)] = ((v[c] - mean) * rs) * g[c] + b[c]
        return carry

    lax.fori_loop(0, _BPW, row, 0)
    pltpu.sync_copy(rows_v, out_hbm.at[pl.ds(base, _BPW)])


def kernel(x, table, gamma, beta):
    mesh = plsc.VectorSubcoreMesh(core_axis_name="c", subcore_axis_name="s")
    f = pl.kernel(
        _body,
        mesh=mesh,
        out_type=jax.ShapeDtypeStruct((_B, _D), jnp.float32),
        scratch_types=[
            pltpu.VMEM((_NCHUNK, _CHUNK), jnp.int32),
            pltpu.VMEM((_BPW, _D), jnp.float32),
            pltpu.VMEM((_D,), jnp.float32),
            pltpu.VMEM((_D,), jnp.float32),
            pltpu.SemaphoreType.DMA,
        ],
        compiler_params=pltpu.CompilerParams(use_tc_tiling_on_sc=False),
    )
    return f(x.astype(jnp.int32), table, gamma, beta)


# speedup vs baseline: 1.1245x; 1.1245x over previous
import functools

import jax
import jax.numpy as jnp
from jax import lax
from jax.experimental import pallas as pl
from jax.experimental.pallas import tpu as pltpu
from jax.experimental.pallas import tpu_sc as plsc

_B = 16384
_D = 64
_EPS = 1e-5
_NW = 32
_BPW = _B // _NW
_CHUNK = 128
_NCHUNK = _BPW // _CHUNK
_NC = _D // 16


def _lanesum(x):
    lanes = lax.iota(jnp.int32, 16)
    for sh in (8, 4, 2, 1):
        x = x + x.at[lanes ^ sh].get(mode="promise_in_bounds")
    return x


def _rsqrt(v):
    i = lax.bitcast_convert_type(v, jnp.int32)
    i = jnp.int32(0x5F3759DF) - lax.shift_right_arithmetic(i, 1)
    y = lax.bitcast_convert_type(i, jnp.float32)
    hv = jnp.float32(0.5) * v
    for _ in range(2):
        y = y * (jnp.float32(1.5) - hv * y * y)
    return y


def _body(x_hbm, table_hbm, gamma_hbm, beta_hbm, out_hbm,
          idx_v, rows_v, gam_v, bet_v, sem, osem):
    wid = lax.axis_index("s") * 2 + lax.axis_index("c")
    base = wid * _BPW

    for j in range(_NCHUNK):
        pltpu.sync_copy(x_hbm.at[pl.ds(base + j * _CHUNK, _CHUNK)],
                        idx_v.at[j])
    pltpu.sync_copy(gamma_hbm, gam_v)
    pltpu.sync_copy(beta_hbm, bet_v)

    copies = [
        pltpu.async_copy(table_hbm.at[idx_v.at[j]],
                         rows_v.at[pl.ds(j * _CHUNK, _CHUNK)], sem)
        for j in range(_NCHUNK)
    ]

    g = [gam_v[pl.ds(c * 16, 16)] for c in range(_NC)]
    b = [bet_v[pl.ds(c * 16, 16)] for c in range(_NC)]
    inv_d = jnp.float32(1.0 / _D)

    def row(r):
        v = [rows_v[r, pl.ds(c * 16, 16)] for c in range(_NC)]
        s = (v[0] + v[1]) + (v[2] + v[3])
        q = (v[0] * v[0] + v[1] * v[1]) + (v[2] * v[2] + v[3] * v[3])
        mean = _lanesum(s) * inv_d
        ex2 = _lanesum(q) * inv_d
        var = ex2 - mean * mean + jnp.float32(_EPS)
        rs = _rsqrt(var)
        for c in range(_NC):
            rows_v[r, pl.ds(c * 16, 16)] = ((v[c] - mean) * rs) * g[c] + b[c]

    out_copies = []
    for j in range(_NCHUNK):
        copies[j].wait()
        plsc.parallel_loop(j * _CHUNK, (j + 1) * _CHUNK, unroll=8)(row)
        out_copies.append(
            pltpu.async_copy(rows_v.at[pl.ds(j * _CHUNK, _CHUNK)],
                             out_hbm.at[pl.ds(base + j * _CHUNK, _CHUNK)],
                             osem))
    for c in out_copies:
        c.wait()


def kernel(x, table, gamma, beta):
    mesh = plsc.VectorSubcoreMesh(core_axis_name="c", subcore_axis_name="s")
    f = pl.kernel(
        _body,
        mesh=mesh,
        out_type=jax.ShapeDtypeStruct((_B, _D), jnp.float32),
        scratch_types=[
            pltpu.VMEM((_NCHUNK, _CHUNK), jnp.int32),
            pltpu.VMEM((_BPW, _D), jnp.float32),
            pltpu.VMEM((_D,), jnp.float32),
            pltpu.VMEM((_D,), jnp.float32),
            pltpu.SemaphoreType.DMA,
            pltpu.SemaphoreType.DMA,
        ],
        compiler_params=pltpu.CompilerParams(use_tc_tiling_on_sc=False),
    )
    return f(x.astype(jnp.int32), table, gamma, beta)


# speedup vs baseline: 1.1634x; 1.0346x over previous
import functools

import jax
import jax.numpy as jnp
from jax import lax
from jax.experimental import pallas as pl
from jax.experimental.pallas import tpu as pltpu
from jax.experimental.pallas import tpu_sc as plsc

_B = 16384
_D = 64
_V = 100000
_EPS = 1e-5
_H = _B // 2
_NT = 16
_FPT = _D // _NT
_NV = _H // 16
_PASS = ((0, 25024), (25024, 25024), (50048, 25024), (75072, _V - 75072))
_NP = len(_PASS)
_BUFW = 25024


def _rsqrt(v):
    i = lax.bitcast_convert_type(v, jnp.int32)
    i = jnp.int32(0x5F3759DF) - lax.shift_right_arithmetic(i, 1)
    y = lax.bitcast_convert_type(i, jnp.float32)
    hv = jnp.float32(0.5) * v
    for _ in range(2):
        y = y * (jnp.float32(1.5) - hv * y * y)
    return y


def _bcast_lane(vec, lane):
    idx = jnp.full((16,), lane, dtype=jnp.int32)
    return vec.at[idx].get(mode="promise_in_bounds")


def _body(x_hbm, tab_hbm, gamma_hbm, beta_hbm, out_hbm,
          idx_v, buf0, buf1, g_v, sum_v, sq_v, part_v, stats_v,
          gam_v, bet_v, sh_part, sh_stats, sem0, sem1):
    cid = lax.axis_index("c")
    sid = lax.axis_index("s")
    bbase = cid * _H

    pltpu.sync_copy(x_hbm.at[pl.ds(bbase, _H)], idx_v)
    pltpu.sync_copy(gamma_hbm, gam_v)
    pltpu.sync_copy(beta_hbm, bet_v)

    bufs = (buf0, buf1)
    sems = (sem0, sem1)

    def start(i):
        k, p = divmod(i, _NP)
        off, sz = _PASS[p]
        f = sid * _FPT + k
        return pltpu.async_copy(tab_hbm.at[f, pl.ds(off, sz)],
                                bufs[i % 2].at[pl.ds(0, sz)], sems[i % 2])

    def run_pass(buf, k, p):
        off, sz = _PASS[p]
        lo = jnp.int32(off)
        hi = jnp.int32(off + sz)

        def vbody(v):
            sl = pl.ds(v * 16, 16)
            iv = idx_v[sl]
            m = (iv >= lo) & (iv < hi)
            loc = jnp.minimum(jnp.maximum(iv - lo, 0), jnp.int32(sz - 1))
            g = plsc.load_gather(buf, [loc], mask=m)
            if p == 0:
                cur = jnp.where(m, g, jnp.float32(0.0))
            else:
                cur = jnp.where(m, g, g_v[sl])
            g_v[sl] = cur
            if p == _NP - 1:
                if k == 0:
                    sum_v[sl] = cur
                    sq_v[sl] = cur * cur
                else:
                    sum_v[sl] = sum_v[sl] + cur
                    sq_v[sl] = sq_v[sl] + cur * cur

        plsc.parallel_loop(0, _NV, unroll=8)(vbody)

    cps = [start(0), None]
    for i in range(_NP * _FPT):
        k, p = divmod(i, _NP)
        if i + 1 < _NP * _FPT:
            cps[(i + 1) % 2] = start(i + 1)
        cps[i % 2].wait()
        run_pass(bufs[i % 2], k, p)
        if p == _NP - 1:
            f = sid * _FPT + k
            pltpu.sync_copy(g_v, out_hbm.at[f, pl.ds(bbase, _H)])

    pltpu.sync_copy(sum_v, sh_part.at[sid, 0])
    pltpu.sync_copy(sq_v, sh_part.at[sid, 1])
    plsc.subcore_barrier()

    st = sid * (_H // _NT)
    inv_d = jnp.float32(1.0 / _D)

    pltpu.sync_copy(sh_part.at[:, 0, pl.ds(st, _H // _NT)], part_v)

    def red_sum(v):
        sl = pl.ds(v * 16, 16)
        acc = part_v[0, sl]
        for t in range(1, _NT):
            acc = acc + part_v[t, sl]
        stats_v[0, pl.ds(st + v * 16, 16)] = acc * inv_d

    plsc.parallel_loop(0, _H // _NT // 16, unroll=4)(red_sum)

    pltpu.sync_copy(sh_part.at[:, 1, pl.ds(st, _H // _NT)], part_v)

    def red_sq(v):
        sl = pl.ds(v * 16, 16)
        acc = part_v[0, sl]
        for t in range(1, _NT):
            acc = acc + part_v[t, sl]
        mean = stats_v[0, pl.ds(st + v * 16, 16)]
        var = acc * inv_d - mean * mean + jnp.float32(_EPS)
        stats_v[1, pl.ds(st + v * 16, 16)] = _rsqrt(var)

    plsc.parallel_loop(0, _H // _NT // 16, unroll=4)(red_sq)

    pltpu.sync_copy(stats_v.at[:, pl.ds(st, _H // _NT)],
                    sh_stats.at[:, pl.ds(st, _H // _NT)])
    plsc.subcore_barrier()
    pltpu.sync_copy(sh_stats, stats_v)

    for k in range(_FPT):
        f = sid * _FPT + k
        pltpu.sync_copy(out_hbm.at[f, pl.ds(bbase, _H)], g_v)
        chunk = (f // 16) * 16
        lane = f - chunk
        gam = _bcast_lane(gam_v[pl.ds(chunk, 16)], lane)
        bet = _bcast_lane(bet_v[pl.ds(chunk, 16)], lane)

        def norm(v):
            sl = pl.ds(v * 16, 16)
            g_v[sl] = ((g_v[sl] - stats_v[0, sl]) * stats_v[1, sl]) * gam + bet

        plsc.parallel_loop(0, _NV, unroll=8)(norm)
        pltpu.sync_copy(g_v, out_hbm.at[f, pl.ds(bbase, _H)])


def kernel(x, table, gamma, beta):
    mesh = plsc.VectorSubcoreMesh(core_axis_name="c", subcore_axis_name="s")
    f = pl.kernel(
        _body,
        mesh=mesh,
        out_type=jax.ShapeDtypeStruct((_D, _B), jnp.float32),
        scratch_types=[
            pltpu.VMEM((_H,), jnp.int32),
            pltpu.VMEM((_BUFW,), jnp.float32),
            pltpu.VMEM((_BUFW,), jnp.float32),
            pltpu.VMEM((_H,), jnp.float32),
            pltpu.VMEM((_H,), jnp.float32),
            pltpu.VMEM((_H,), jnp.float32),
            pltpu.VMEM((_NT, _H // _NT), jnp.float32),
            pltpu.VMEM((2, _H), jnp.float32),
            pltpu.VMEM((_D,), jnp.float32),
            pltpu.VMEM((_D,), jnp.float32),
            pltpu.VMEM_SHARED((_NT, 2, _H), jnp.float32),
            pltpu.VMEM_SHARED((2, _H), jnp.float32),
            pltpu.SemaphoreType.DMA,
            pltpu.SemaphoreType.DMA,
        ],
        compiler_params=pltpu.CompilerParams(use_tc_tiling_on_sc=False,
                                             needs_layout_passes=False),
    )
    out_t = f(x.astype(jnp.int32), table.T, gamma, beta)
    return out_t.T
